# Initial kernel scaffold; baseline (speedup 1.0000x reference)
import functools

import jax
import jax.numpy as jnp
from jax import lax
from jax.experimental import pallas as pl
from jax.experimental.pallas import tpu as pltpu
from jax.experimental.pallas import tpu_sc as plsc

_BASE_VOCAB = 1000000
_EMBED_DIM = 64
_NUM_SEGMENTS = 250000
_N = 1000000

_NW = 32
_SEG_TILE = 625
_NTILES = _NUM_SEGMENTS // _SEG_TILE
_CHUNK = 128
_TILES_PER_W = (_NTILES + _NW - 1) // _NW
_STARTS_PAD = 408


def _sc_kernel(table, eidx, sidx, starts, out, starts_v, idx_v, seg_v,
               rows_v, acc, cnt, gsem):
    wid = lax.axis_index("s") * 2 + lax.axis_index("c")
    pltpu.sync_copy(starts.at[pl.ds(0, _STARTS_PAD)], starts_v)

    ones = jnp.full((16,), 1.0, dtype=jnp.float32)
    zeros = jnp.zeros((16,), dtype=jnp.float32)

    def tile_body(i, _):
        t = wid + i * _NW

        @pl.when(t < _NTILES)
        def _():
            r_lo = starts_v[t]
            r_hi = starts_v[t + 1]
            r0 = lax.bitwise_and(r_lo, jnp.int32(-8))
            nchunks = (r_hi - r0 + (_CHUNK - 1)) // _CHUNK
            seg_lo = t * _SEG_TILE

            def zero_body(s, _):
                for c in range(4):
                    acc[s, pl.ds(c * 16, 16)] = zeros
                cnt[s, :] = zeros
                return 0

            lax.fori_loop(0, _SEG_TILE + 2, zero_body, 0)

            def chunk_body(k, _):
                base = r0 + k * _CHUNK
                pltpu.sync_copy(eidx.at[pl.ds(base, _CHUNK)], idx_v)
                pltpu.sync_copy(sidx.at[pl.ds(base, _CHUNK)], seg_v)
                pltpu.async_copy(table.at[idx_v], rows_v, gsem).wait()

                def row_body(j, _):
                    seg = seg_v[j]
                    ls = seg - seg_lo
                    lsc = jnp.minimum(jnp.maximum(ls, -1), _SEG_TILE) + 1
                    for c in range(4):
                        acc[lsc, pl.ds(c * 16, 16)] += rows_v[j, pl.ds(c * 16, 16)]
                    cnt[lsc, :] += ones
                    return 0

                lax.fori_loop(0, _CHUNK, row_body, 0)
                return 0

            lax.fori_loop(0, nchunks, chunk_body, 0)

            def fin_body(s, _):
                inv = 1.0 / jnp.maximum(cnt[s + 1, :], 1.0)
                for c in range(4):
                    acc[s + 1, pl.ds(c * 16, 16)] *= inv
                return 0

            lax.fori_loop(0, _SEG_TILE, fin_body, 0)
            pltpu.sync_copy(acc.at[pl.ds(1, _SEG_TILE)],
                            out.at[pl.ds(seg_lo, _SEG_TILE)])

        return 0

    lax.fori_loop(0, _TILES_PER_W, tile_body, 0)


def kernel(base_embeddings, extract_index, scatter_index):
    bounds = jnp.arange(0, _NUM_SEGMENTS + 1, _SEG_TILE, dtype=jnp.int32)
    starts = jnp.searchsorted(scatter_index, bounds).astype(jnp.int32)
    starts = jnp.concatenate(
        [starts, jnp.full((_STARTS_PAD - _NTILES - 1,), _N, jnp.int32)])
    eidx = jnp.concatenate(
        [extract_index, jnp.zeros((_CHUNK,), jnp.int32)])
    sidx = jnp.concatenate(
        [scatter_index, jnp.full((_CHUNK,), _NUM_SEGMENTS, jnp.int32)])

    mesh = plsc.VectorSubcoreMesh(core_axis_name="c", subcore_axis_name="s")
    f = functools.partial(
        pl.kernel,
        mesh=mesh,
        out_type=jax.ShapeDtypeStruct((_NUM_SEGMENTS, _EMBED_DIM),
                                      jnp.float32),
        scratch_types=[
            pltpu.VMEM((_STARTS_PAD,), jnp.int32),
            pltpu.VMEM((_CHUNK,), jnp.int32),
            pltpu.VMEM((_CHUNK,), jnp.int32),
            pltpu.VMEM((_CHUNK, _EMBED_DIM), jnp.float32),
            pltpu.VMEM((_SEG_TILE + 2, _EMBED_DIM), jnp.float32),
            pltpu.VMEM((_SEG_TILE + 2, 16), jnp.float32),
            pltpu.SemaphoreType.DMA,
        ],
    )(_sc_kernel)
    return f(base_embeddings, eidx, sidx, starts)


# speedup vs baseline: 2.2234x; 2.2234x over previous
import functools

import jax
import jax.numpy as jnp
from jax import lax
from jax.experimental import pallas as pl
from jax.experimental.pallas import tpu as pltpu
from jax.experimental.pallas import tpu_sc as plsc

_BASE_VOCAB = 1000000
_EMBED_DIM = 64
_NUM_SEGMENTS = 250000
_N = 1000000

_NW = 32
_SEG_TILE = 1000
_NTILES = _NUM_SEGMENTS // _SEG_TILE
_CHUNK = 128
_TILES_PER_W = (_NTILES + _NW - 1) // _NW
_STARTS_PAD = 272


def _sc_kernel(table, eidx, sidx, starts, out, starts_v, idx_v, seg_v,
               rows_v, acc, cnt, gsem):
    wid = lax.axis_index("s") * 2 + lax.axis_index("c")
    pltpu.sync_copy(starts.at[pl.ds(0, _STARTS_PAD)], starts_v)

    ones = jnp.full((16,), 1.0, dtype=jnp.float32)
    zeros = jnp.zeros((16,), dtype=jnp.float32)

    def tile_body(i, _):
        t = wid + i * _NW

        @pl.when(t < _NTILES)
        def _():
            sv = starts_v[pl.ds(t, 16)]
            r_lo = sv[0]
            r_hi = sv[1]
            r0 = lax.bitwise_and(r_lo, jnp.int32(-8))
            nchunks = (r_hi - r0 + (_CHUNK - 1)) // _CHUNK
            seg_lo = t * _SEG_TILE

            def zero_body(s, _):
                for c in range(4):
                    acc[s, pl.ds(c * 16, 16)] = zeros
                cnt[s, :] = zeros
                return 0

            lax.fori_loop(0, _SEG_TILE + 1, zero_body, 0)

            def chunk_body(k, _):
                base = pl.multiple_of(r0 + k * _CHUNK, 8)
                pltpu.sync_copy(eidx.at[pl.ds(base, _CHUNK)], idx_v)
                pltpu.sync_copy(sidx.at[pl.ds(base, _CHUNK)], seg_v)
                pltpu.async_copy(table.at[idx_v], rows_v, gsem).wait()

                def grp_body(g, _):
                    segs = seg_v[pl.ds(g * 16, 16)]
                    for lane in range(16):
                        seg = segs[lane]
                        ls = seg - seg_lo
                        lsc = jnp.where(
                            (ls >= 0) & (ls < _SEG_TILE), ls, _SEG_TILE)
                        j = g * 16 + lane
                        for c in range(4):
                            acc[lsc, pl.ds(c * 16, 16)] += (
                                rows_v[j, pl.ds(c * 16, 16)])
                        cnt[lsc, :] += ones
                    return 0

                lax.fori_loop(0, _CHUNK // 16, grp_body, 0)
                return 0

            lax.fori_loop(0, nchunks, chunk_body, 0)

            def fin_body(s, _):
                inv = 1.0 / jnp.maximum(cnt[s, :], 1.0)
                for c in range(4):
                    acc[s, pl.ds(c * 16, 16)] *= inv
                return 0

            lax.fori_loop(0, _SEG_TILE, fin_body, 0)
            pltpu.sync_copy(acc.at[pl.ds(0, _SEG_TILE)],
                            out.at[pl.ds(seg_lo, _SEG_TILE)])

        return 0

    lax.fori_loop(0, _TILES_PER_W, tile_body, 0)


def kernel(base_embeddings, extract_index, scatter_index):
    bounds = jnp.arange(0, _NUM_SEGMENTS + 1, _SEG_TILE, dtype=jnp.int32)
    starts = jnp.searchsorted(scatter_index, bounds).astype(jnp.int32)
    starts = jnp.concatenate(
        [starts, jnp.full((_STARTS_PAD - _NTILES - 1,), _N, jnp.int32)])

    eidx = jnp.concatenate(
        [extract_index, jnp.zeros((_CHUNK,), jnp.int32)])
    sidx = jnp.concatenate(
        [scatter_index, jnp.full((_CHUNK,), _NUM_SEGMENTS, jnp.int32)])

    mesh = plsc.VectorSubcoreMesh(core_axis_name="c", subcore_axis_name="s")
    f = functools.partial(
        pl.kernel,
        mesh=mesh,
        out_type=jax.ShapeDtypeStruct((_NUM_SEGMENTS, _EMBED_DIM),
                                      jnp.float32),
        scratch_types=[
            pltpu.VMEM((_STARTS_PAD,), jnp.int32),
            pltpu.VMEM((_CHUNK,), jnp.int32),
            pltpu.VMEM((_CHUNK,), jnp.int32),
            pltpu.VMEM((_CHUNK, _EMBED_DIM), jnp.float32),
            pltpu.VMEM((_SEG_TILE + 1, _EMBED_DIM), jnp.float32),
            pltpu.VMEM((_SEG_TILE + 1, 16), jnp.float32),
            pltpu.SemaphoreType.DMA,
        ],
        compiler_params=pltpu.CompilerParams(use_tc_tiling_on_sc=False),
    )(_sc_kernel)
    return f(base_embeddings, eidx, sidx, starts)


# speedup vs baseline: 2.2256x; 1.0010x over previous
import functools

import jax
import jax.numpy as jnp
from jax import lax
from jax.experimental import pallas as pl
from jax.experimental.pallas import tpu as pltpu
from jax.experimental.pallas import tpu_sc as plsc

_BASE_VOCAB = 1000000
_EMBED_DIM = 64
_NUM_SEGMENTS = 250000
_N = 1000000

_NW = 32
_SEG_TILE = 1000
_NTILES = _NUM_SEGMENTS // _SEG_TILE
_CHUNK = 128
_TILES_PER_W = (_NTILES + _NW - 1) // _NW
_STARTS_PAD = 272


def _sc_kernel(table, eidx, sidx, starts, out, starts_v, idx_v, seg_v,
               rows_v, acc, cnt, gsem):
    wid = lax.axis_index("s") * 2 + lax.axis_index("c")
    pltpu.sync_copy(starts.at[pl.ds(0, _STARTS_PAD)], starts_v)

    ones = jnp.full((16,), 1.0, dtype=jnp.float32)
    zeros = jnp.zeros((16,), dtype=jnp.float32)

    def tile_body(i, _):
        t = wid + i * _NW

        @pl.when(t < _NTILES)
        def _():
            sv = starts_v[pl.ds(t, 16)]
            r_lo = sv[0]
            r_hi = sv[1]
            r0 = lax.bitwise_and(r_lo, jnp.int32(-8))
            nchunks = (r_hi - r0 + (_CHUNK - 1)) // _CHUNK
            seg_lo = t * _SEG_TILE

            def zero_body(s, _):
                for c in range(4):
                    acc[s, pl.ds(c * 16, 16)] = zeros
                cnt[s, :] = zeros
                return 0

            lax.fori_loop(0, _SEG_TILE + 1, zero_body, 0)

            def chunk_body(k, _):
                base_u = r0 + k * _CHUNK
                base = pl.multiple_of(
                    jnp.minimum(base_u, _N - _CHUNK), 8)
                shift = base_u - base
                pltpu.sync_copy(eidx.at[pl.ds(base, _CHUNK)], idx_v)
                pltpu.sync_copy(sidx.at[pl.ds(base, _CHUNK)], seg_v)
                pltpu.async_copy(table.at[idx_v], rows_v, gsem).wait()

                def grp_body(g, _):
                    segs = seg_v[pl.ds(g * 16, 16)]
                    for lane in range(16):
                        seg = segs[lane]
                        ls = seg - seg_lo
                        j = g * 16 + lane
                        lsc = jnp.where(
                            (ls >= 0) & (ls < _SEG_TILE) & (j >= shift),
                            ls, _SEG_TILE)
                        for c in range(4):
                            acc[lsc, pl.ds(c * 16, 16)] += (
                                rows_v[j, pl.ds(c * 16, 16)])
                        cnt[lsc, :] += ones
                    return 0

                lax.fori_loop(0, _CHUNK // 16, grp_body, 0)
                return 0

            lax.fori_loop(0, nchunks, chunk_body, 0)

            def fin_body(s, _):
                inv = 1.0 / jnp.maximum(cnt[s, :], 1.0)
                for c in range(4):
                    acc[s, pl.ds(c * 16, 16)] *= inv
                return 0

            lax.fori_loop(0, _SEG_TILE, fin_body, 0)
            pltpu.sync_copy(acc.at[pl.ds(0, _SEG_TILE)],
                            out.at[pl.ds(seg_lo, _SEG_TILE)])

        return 0

    lax.fori_loop(0, _TILES_PER_W, tile_body, 0)


def kernel(base_embeddings, extract_index, scatter_index):
    bounds = jnp.arange(0, _STARTS_PAD * _SEG_TILE, _SEG_TILE,
                        dtype=jnp.int32)
    starts = jnp.searchsorted(scatter_index, bounds).astype(jnp.int32)
    eidx = extract_index
    sidx = scatter_index

    mesh = plsc.VectorSubcoreMesh(core_axis_name="c", subcore_axis_name="s")
    f = functools.partial(
        pl.kernel,
        mesh=mesh,
        out_type=jax.ShapeDtypeStruct((_NUM_SEGMENTS, _EMBED_DIM),
                                      jnp.float32),
        scratch_types=[
            pltpu.VMEM((_STARTS_PAD,), jnp.int32),
            pltpu.VMEM((_CHUNK,), jnp.int32),
            pltpu.VMEM((_CHUNK,), jnp.int32),
            pltpu.VMEM((_CHUNK, _EMBED_DIM), jnp.float32),
            pltpu.VMEM((_SEG_TILE + 1, _EMBED_DIM), jnp.float32),
            pltpu.VMEM((_SEG_TILE + 1, 16), jnp.float32),
            pltpu.SemaphoreType.DMA,
        ],
        compiler_params=pltpu.CompilerParams(use_tc_tiling_on_sc=False),
    )(_sc_kernel)
    return f(base_embeddings, eidx, sidx, starts)


# speedup vs baseline: 3.3981x; 1.5268x over previous
import functools

import jax
import jax.numpy as jnp
from jax import lax
from jax.experimental import pallas as pl
from jax.experimental.pallas import tpu as pltpu
from jax.experimental.pallas import tpu_sc as plsc

_BASE_VOCAB = 1000000
_EMBED_DIM = 64
_NUM_SEGMENTS = 250000
_N = 1000000

_NW = 32
_SEG_TILE = 1000
_NTILES = _NUM_SEGMENTS // _SEG_TILE
_CHUNK = 128
_TILES_PER_W = (_NTILES + _NW - 1) // _NW
_STARTS_PAD = 272
_NBUF = 4


def _sc_kernel(table, eidx, sidx, starts, out, starts_v, idx_v, seg_v,
               rows_v, acc, cnt, isem, ssem, gsem):
    wid = lax.axis_index("s") * 2 + lax.axis_index("c")
    pltpu.sync_copy(starts.at[pl.ds(0, _STARTS_PAD)], starts_v)

    ones = jnp.full((16,), 1.0, dtype=jnp.float32)
    zeros = jnp.zeros((16,), dtype=jnp.float32)

    def tile_body(i, _):
        t = wid + i * _NW

        @pl.when(t < _NTILES)
        def _():
            sv = starts_v[pl.ds(t, 16)]
            r_lo = sv[0]
            r_hi = sv[1]
            r0 = lax.bitwise_and(r_lo, jnp.int32(-8))
            nchunks = (r_hi - r0 + (_CHUNK - 1)) // _CHUNK
            seg_lo = t * _SEG_TILE

            def chunk_base(k):
                return pl.multiple_of(
                    jnp.minimum(r0 + k * _CHUNK, _N - _CHUNK), 8)

            def start_idx(k):
                s = lax.rem(k, _NBUF)
                b = chunk_base(k)
                pltpu.async_copy(eidx.at[pl.ds(b, _CHUNK)], idx_v.at[s],
                                 isem.at[s])
                pltpu.async_copy(sidx.at[pl.ds(b, _CHUNK)], seg_v.at[s],
                                 ssem.at[s])

            def start_gather(k):
                s = lax.rem(k, _NBUF)
                b = chunk_base(k)
                pltpu.make_async_copy(eidx.at[pl.ds(b, _CHUNK)],
                                      idx_v.at[s], isem.at[s]).wait()
                pltpu.async_copy(table.at[idx_v.at[s]], rows_v.at[s],
                                 gsem.at[s])

            @pl.when(nchunks > 0)
            def _():
                start_idx(0)

                @pl.when(nchunks > 1)
                def _():
                    start_idx(1)

            def zero_body(s, _):
                for c in range(4):
                    acc[s, pl.ds(c * 16, 16)] = zeros
                cnt[s, :] = zeros
                return 0

            lax.fori_loop(0, _SEG_TILE + 1, zero_body, 0)

            @pl.when(nchunks > 0)
            def _():
                start_gather(0)

            def chunk_body(k, _):
                s = lax.rem(k, _NBUF)
                b = chunk_base(k)
                shift = (r0 + k * _CHUNK) - b

                @pl.when(k + 1 < nchunks)
                def _():
                    start_gather(k + 1)

                @pl.when(k + 2 < nchunks)
                def _():
                    start_idx(k + 2)

                pltpu.make_async_copy(table.at[idx_v.at[s]], rows_v.at[s],
                                      gsem.at[s]).wait()
                pltpu.make_async_copy(sidx.at[pl.ds(b, _CHUNK)],
                                      seg_v.at[s], ssem.at[s]).wait()

                def grp_body(g, _):
                    segs = seg_v[s, pl.ds(g * 16, 16)]
                    for lane in range(16):
                        seg = segs[lane]
                        ls = seg - seg_lo
                        j = g * 16 + lane
                        lsc = jnp.where(
                            (ls >= 0) & (ls < _SEG_TILE) & (j >= shift),
                            ls, _SEG_TILE)
                        for c in range(4):
                            plsc.addupdate(
                                acc.at[lsc, pl.ds(c * 16, 16)],
                                rows_v[s, j, pl.ds(c * 16, 16)])
                        plsc.addupdate(cnt.at[lsc], ones)
                    return 0

                lax.fori_loop(0, _CHUNK // 16, grp_body, 0)
                return 0

            lax.fori_loop(0, nchunks, chunk_body, 0)

            def fin_body(s, _):
                inv = 1.0 / jnp.maximum(cnt[s, :], 1.0)
                for c in range(4):
                    acc[s, pl.ds(c * 16, 16)] *= inv
                return 0

            lax.fori_loop(0, _SEG_TILE, fin_body, 0)
            pltpu.sync_copy(acc.at[pl.ds(0, _SEG_TILE)],
                            out.at[pl.ds(seg_lo, _SEG_TILE)])

        return 0

    lax.fori_loop(0, _TILES_PER_W, tile_body, 0)


def kernel(base_embeddings, extract_index, scatter_index):
    bounds = jnp.arange(0, _STARTS_PAD * _SEG_TILE, _SEG_TILE,
                        dtype=jnp.int32)
    starts = jnp.searchsorted(scatter_index, bounds).astype(jnp.int32)

    mesh = plsc.VectorSubcoreMesh(core_axis_name="c", subcore_axis_name="s")
    f = functools.partial(
        pl.kernel,
        mesh=mesh,
        out_type=jax.ShapeDtypeStruct((_NUM_SEGMENTS, _EMBED_DIM),
                                      jnp.float32),
        scratch_types=[
            pltpu.VMEM((_STARTS_PAD,), jnp.int32),
            pltpu.VMEM((_NBUF, _CHUNK), jnp.int32),
            pltpu.VMEM((_NBUF, _CHUNK), jnp.int32),
            pltpu.VMEM((_NBUF, _CHUNK, _EMBED_DIM), jnp.float32),
            pltpu.VMEM((_SEG_TILE + 1, _EMBED_DIM), jnp.float32),
            pltpu.VMEM((_SEG_TILE + 1, 16), jnp.float32),
            pltpu.SemaphoreType.DMA((_NBUF,)),
            pltpu.SemaphoreType.DMA((_NBUF,)),
            pltpu.SemaphoreType.DMA((_NBUF,)),
        ],
        compiler_params=pltpu.CompilerParams(use_tc_tiling_on_sc=False),
    )(_sc_kernel)
    return f(base_embeddings, extract_index, scatter_index, starts)
